# Initial kernel scaffold; baseline (speedup 1.0000x reference)
import jax
import jax.numpy as jnp
from jax.experimental import pallas as pl


def kernel(x, enc_W0, enc_b0, enc_W1, enc_b1, enc_W2, enc_b2, enc_W3, enc_b3, dec_W0, dec_b0, dec_W1, dec_b1, dec_W2, dec_b2, dec_W3, dec_b3, cb0, cb1, cb2):
    raise NotImplementedError("write your pallas kernel here")


# speedup vs baseline: 1.3609x; 1.3609x over previous
import jax
import jax.numpy as jnp
from jax.experimental import pallas as pl

_K = 1024
_NUM_CAT = 18


def _dot(a, b):
    return jax.lax.dot_general(a, b, (((1,), (0,)), ((), ())),
                               preferred_element_type=jnp.float32)


def _dot_rhs_t(a, b):
    return jax.lax.dot_general(a, b, (((1,), (1,)), ((), ())),
                               preferred_element_type=jnp.float32)


def _dot_exact(a, b):
    return jax.lax.dot_general(a, b, (((1,), (0,)), ((), ())),
                               preferred_element_type=jnp.float32,
                               precision=jax.lax.Precision.HIGHEST)


def _fused_body(x_ref,
                ew0, eb0, ew1, eb1, ew2, eb2, ew3, eb3,
                dw0, db0, dw1, db1, dw2, db2, dw3, db3,
                cb0r, cb1r, cb2r, cbt0, cbt1, cbt2,
                keys_ref, n0_ref, n1_ref, n2_ref,
                recon_ref, qloss_ref):
    i = pl.program_id(0)
    x = x_ref[...]

    h = jnp.maximum(_dot(x, ew0[...]) + eb0[...], 0.0)
    h = jnp.maximum(_dot(h, ew1[...]) + eb1[...], 0.0)
    h = jnp.maximum(_dot(h, ew2[...]) + eb2[...], 0.0)
    z = _dot(h, ew3[...]) + eb3[...]

    res = z
    qrows = jnp.zeros((z.shape[0], 1), jnp.float32)
    key = jnp.zeros((z.shape[0], 1), jnp.int32)
    for cb_ref, cbt_ref, n_ref in ((cb0r, cbt0, n0_ref),
                                   (cb1r, cbt1, n1_ref),
                                   (cb2r, cbt2, n2_ref)):
        cb = cb_ref[...]
        cbt = cbt_ref[...]
        cbn = jnp.sum(cbt * cbt, axis=0, keepdims=True)
        scores = _dot_rhs_t(res, cb)
        rn = jnp.sum(res * res, axis=1, keepdims=True)
        d = (rn - 2.0 * scores) + cbn
        mind = jnp.min(d, axis=1, keepdims=True)
        iota = jax.lax.broadcasted_iota(jnp.int32, d.shape, 1)
        ids = jnp.min(jnp.where(d == mind, iota, _K), axis=1, keepdims=True)
        onehot = (iota == ids).astype(jnp.float32)
        emb = _dot_exact(onehot, cb)
        diff = res - emb
        qrows = qrows + 1.25 * jnp.sum(diff * diff, axis=1, keepdims=True)
        n_ref[...] = jnp.sqrt(jnp.sum(emb * emb, axis=1, keepdims=True))
        res = diff
        key = key * _K + ids
    keys_ref[...] = key

    emb_sum = z - res

    h = jnp.maximum(_dot(emb_sum, dw0[...]) + db0[...], 0.0)
    h = jnp.maximum(_dot(h, dw1[...]) + db1[...], 0.0)
    h = jnp.maximum(_dot(h, dw2[...]) + db2[...], 0.0)
    h = _dot(h, dw3[...]) + db3[...]
    nrm = jnp.sqrt(jnp.sum(h * h, axis=1, keepdims=True))
    xh = h / (nrm + 1e-12)
    col = jax.lax.broadcasted_iota(jnp.int32, xh.shape, 1)
    head = col < (xh.shape[1] - _NUM_CAT)
    xhh = jnp.where(head, xh, 0.0)
    hn = jnp.sqrt(jnp.sum(xhh * xhh, axis=1, keepdims=True))
    xh2 = jnp.where(head, xh / (hn + 1e-12), xh)
    rrows = jnp.sum((xh2 - x) ** 2, axis=1, keepdims=True)

    @pl.when(i == 0)
    def _init():
        recon_ref[...] = jnp.zeros_like(recon_ref)
        qloss_ref[...] = jnp.zeros_like(qloss_ref)

    recon_ref[...] += jnp.sum(rrows, keepdims=True)
    qloss_ref[...] += jnp.sum(qrows, keepdims=True)


def _uniq_body(kc_ref, kr_ref, out_ref):
    c = pl.program_id(0)
    rows = kc_ref[...]
    cols = kr_ref[...]
    shape = (rows.shape[0], cols.shape[1])
    eq = rows == cols
    colid = jax.lax.broadcasted_iota(jnp.int32, shape, 1)
    rowid = c * rows.shape[0] + jax.lax.broadcasted_iota(jnp.int32, shape, 0)
    dup = jnp.logical_and(eq, colid > rowid)
    hasdup = jnp.any(dup, axis=1, keepdims=True)

    @pl.when(c == 0)
    def _init():
        out_ref[...] = jnp.zeros_like(out_ref)

    out_ref[...] += jnp.sum(hasdup.astype(jnp.float32), keepdims=True)


def kernel(x, enc_W0, enc_b0, enc_W1, enc_b1, enc_W2, enc_b2, enc_W3, enc_b3,
           dec_W0, dec_b0, dec_W1, dec_b1, dec_W2, dec_b2, dec_W3, dec_b3,
           cb0, cb1, cb2):
    B, D = x.shape
    R = 512
    grid = (B // R,)

    ebs = [b.reshape(1, -1) for b in (enc_b0, enc_b1, enc_b2, enc_b3)]
    dbs = [b.reshape(1, -1) for b in (dec_b0, dec_b1, dec_b2, dec_b3)]
    cbts = [c.T for c in (cb0, cb1, cb2)]

    full = lambda a: pl.BlockSpec(a.shape, lambda i: (0,) * a.ndim)
    in_specs = [pl.BlockSpec((R, D), lambda i: (i, 0))]
    args = [x]
    for W, b in zip((enc_W0, enc_W1, enc_W2, enc_W3), ebs):
        in_specs += [full(W), full(b)]
        args += [W, b]
    for W, b in zip((dec_W0, dec_W1, dec_W2, dec_W3), dbs):
        in_specs += [full(W), full(b)]
        args += [W, b]
    for c in (cb0, cb1, cb2):
        in_specs.append(full(c))
        args.append(c)
    for c in cbts:
        in_specs.append(full(c))
        args.append(c)

    col_spec = pl.BlockSpec((R, 1), lambda i: (i, 0))
    acc_spec = pl.BlockSpec((1, 1), lambda i: (0, 0))
    out_shape = [
        jax.ShapeDtypeStruct((B, 1), jnp.int32),
        jax.ShapeDtypeStruct((B, 1), jnp.float32),
        jax.ShapeDtypeStruct((B, 1), jnp.float32),
        jax.ShapeDtypeStruct((B, 1), jnp.float32),
        jax.ShapeDtypeStruct((1, 1), jnp.float32),
        jax.ShapeDtypeStruct((1, 1), jnp.float32),
    ]
    out_specs = [col_spec, col_spec, col_spec, col_spec, acc_spec, acc_spec]

    keys, n0, n1, n2, recon_s, qloss_s = pl.pallas_call(
        _fused_body,
        grid=grid,
        in_specs=in_specs,
        out_specs=out_specs,
        out_shape=out_shape,
    )(*args)

    C = 256
    dup_cnt = pl.pallas_call(
        _uniq_body,
        grid=(B // C,),
        in_specs=[pl.BlockSpec((C, 1), lambda c: (c, 0)),
                  pl.BlockSpec((1, B), lambda c: (0, 0))],
        out_specs=pl.BlockSpec((1, 1), lambda c: (0, 0)),
        out_shape=jax.ShapeDtypeStruct((1, 1), jnp.float32),
    )(keys, keys.reshape(1, B))

    recon_mean = recon_s[0, 0] / B
    qloss_mean = qloss_s[0, 0] / B
    loss = recon_mean + qloss_mean
    embs_norm = jnp.concatenate([n0, n1, n2], axis=1)
    p_unique = (B - dup_cnt[0, 0]) / B
    return (loss, recon_mean, qloss_mean, embs_norm, p_unique)


# speedup vs baseline: 2.6103x; 1.9180x over previous
import jax
import jax.numpy as jnp
from jax.experimental import pallas as pl
from jax.experimental.pallas import tpu as pltpu

_K = 1024
_NUM_CAT = 18


def _dot(a, b):
    return jax.lax.dot_general(a, b, (((1,), (0,)), ((), ())),
                               preferred_element_type=jnp.float32)


def _dot_rhs_t(a, b):
    return jax.lax.dot_general(a, b, (((1,), (1,)), ((), ())),
                               preferred_element_type=jnp.float32)


def _fused_body(x_ref,
                ew0, eb0, ew1, eb1, ew2, eb2, ew3, eb3,
                dw0, db0, dw1, db1, dw2, db2, dw3, db3,
                cb0r, cb1r, cb2r,
                emb3_ref, recon_ref, qloss_ref, dup_ref,
                cbt0_s, cbt1_s, cbt2_s,
                s01_0, s01_1, s01_2, s2_0, s2_1, s2_2,
                krow_s):
    i = pl.program_id(0)
    R = x_ref.shape[0]

    @pl.when(i == 0)
    def _prep():
        for cb_ref, cbt_s, s01_s, s2_s in ((cb0r, cbt0_s, s01_0, s2_0),
                                           (cb1r, cbt1_s, s01_1, s2_1),
                                           (cb2r, cbt2_s, s01_2, s2_2)):
            cb = cb_ref[...]
            cbt_s[...] = jnp.transpose(cb)
            s0 = cb.astype(jnp.bfloat16).astype(jnp.float32)
            r1 = cb - s0
            s1 = r1.astype(jnp.bfloat16).astype(jnp.float32)
            s01_s[...] = jnp.concatenate([s0, s1], axis=1)
            s2_s[...] = r1 - s1
        recon_ref[...] = jnp.zeros_like(recon_ref)
        qloss_ref[...] = jnp.zeros_like(qloss_ref)
        dup_ref[...] = jnp.zeros_like(dup_ref)

    x = x_ref[...]

    h = jnp.maximum(_dot(x, ew0[...]) + eb0[...], 0.0)
    h = jnp.maximum(_dot(h, ew1[...]) + eb1[...], 0.0)
    h = jnp.maximum(_dot(h, ew2[...]) + eb2[...], 0.0)
    z = _dot(h, ew3[...]) + eb3[...]

    res = z
    qrows = jnp.zeros((R, 1), jnp.float32)
    key = jnp.zeros((R, 1), jnp.int32)
    norms = []
    for cb_ref, cbt_s, s01_s, s2_s in ((cb0r, cbt0_s, s01_0, s2_0),
                                       (cb1r, cbt1_s, s01_1, s2_1),
                                       (cb2r, cbt2_s, s01_2, s2_2)):
        cbt = cbt_s[...]
        cbn = jnp.sum(cbt * cbt, axis=0, keepdims=True)
        scores = _dot_rhs_t(res, cb_ref[...])
        rn = jnp.sum(res * res, axis=1, keepdims=True)
        d = (rn - 2.0 * scores) + cbn
        ids = jnp.argmin(d, axis=1).reshape(-1, 1)
        iota = jax.lax.broadcasted_iota(jnp.int32, d.shape, 1)
        onehot = (iota == ids).astype(jnp.float32)
        p01 = _dot(onehot, s01_s[...])
        emb = (p01[:, :64] + p01[:, 64:]) + _dot(onehot, s2_s[...])
        diff = res - emb
        qrows = qrows + 1.25 * jnp.sum(diff * diff, axis=1, keepdims=True)
        norms.append(jnp.sqrt(jnp.sum(emb * emb, axis=1, keepdims=True)))
        res = diff
        key = key * _K + ids
    emb3_ref[...] = jnp.concatenate(norms, axis=1)

    kf = jax.lax.bitcast_convert_type(key, jnp.float32)
    krow_s[0:1, pl.ds(i * R, R)] = jax.lax.bitcast_convert_type(
        jnp.transpose(kf), jnp.int32)
    rowg = i * R + jax.lax.broadcasted_iota(jnp.int32, (R, _K), 0)
    colg0 = jax.lax.broadcasted_iota(jnp.int32, (R, _K), 1)

    hasdup = jnp.zeros((R, 1), jnp.bool_)
    for c in range(krow_s.shape[1] // _K):
        cols = krow_s[0:1, c * _K:(c + 1) * _K]
        m = jnp.logical_and(key == cols, (c * _K + colg0) < rowg)
        hasdup = jnp.logical_or(hasdup, jnp.any(m, axis=1, keepdims=True))
    dup_ref[...] += jnp.sum(hasdup.astype(jnp.float32), keepdims=True)

    emb_sum = z - res

    h = jnp.maximum(_dot(emb_sum, dw0[...]) + db0[...], 0.0)
    h = jnp.maximum(_dot(h, dw1[...]) + db1[...], 0.0)
    h = jnp.maximum(_dot(h, dw2[...]) + db2[...], 0.0)
    h = _dot(h, dw3[...]) + db3[...]
    nrm = jnp.sqrt(jnp.sum(h * h, axis=1, keepdims=True))
    xh = h / (nrm + 1e-12)
    col = jax.lax.broadcasted_iota(jnp.int32, xh.shape, 1)
    head = col < (xh.shape[1] - _NUM_CAT)
    xhh = jnp.where(head, xh, 0.0)
    hn = jnp.sqrt(jnp.sum(xhh * xhh, axis=1, keepdims=True))
    xh2 = jnp.where(head, xh / (hn + 1e-12), xh)
    rrows = jnp.sum((xh2 - x) ** 2, axis=1, keepdims=True)

    recon_ref[...] += jnp.sum(rrows, keepdims=True)
    qloss_ref[...] += jnp.sum(qrows, keepdims=True)


def kernel(x, enc_W0, enc_b0, enc_W1, enc_b1, enc_W2, enc_b2, enc_W3, enc_b3,
           dec_W0, dec_b0, dec_W1, dec_b1, dec_W2, dec_b2, dec_W3, dec_b3,
           cb0, cb1, cb2):
    B, D = x.shape
    R = 1024
    grid = (B // R,)

    ebs = [b.reshape(1, -1) for b in (enc_b0, enc_b1, enc_b2, enc_b3)]
    dbs = [b.reshape(1, -1) for b in (dec_b0, dec_b1, dec_b2, dec_b3)]

    full = lambda a: pl.BlockSpec(a.shape, lambda i: (0,) * a.ndim)
    in_specs = [pl.BlockSpec((R, D), lambda i: (i, 0))]
    args = [x]
    for W, b in zip((enc_W0, enc_W1, enc_W2, enc_W3), ebs):
        in_specs += [full(W), full(b)]
        args += [W, b]
    for W, b in zip((dec_W0, dec_W1, dec_W2, dec_W3), dbs):
        in_specs += [full(W), full(b)]
        args += [W, b]
    for c in (cb0, cb1, cb2):
        in_specs.append(full(c))
        args.append(c)

    acc_spec = pl.BlockSpec((1, 1), lambda i: (0, 0))
    out_shape = [
        jax.ShapeDtypeStruct((B, 3), jnp.float32),
        jax.ShapeDtypeStruct((1, 1), jnp.float32),
        jax.ShapeDtypeStruct((1, 1), jnp.float32),
        jax.ShapeDtypeStruct((1, 1), jnp.float32),
    ]
    out_specs = [pl.BlockSpec((R, 3), lambda i: (i, 0)),
                 acc_spec, acc_spec, acc_spec]
    K, L = cb0.shape
    scratch_shapes = (
        [pltpu.VMEM((L, K), jnp.float32)] * 3
        + [pltpu.VMEM((K, 2 * L), jnp.float32)] * 3
        + [pltpu.VMEM((K, L), jnp.float32)] * 3
        + [pltpu.VMEM((1, B), jnp.int32)]
    )

    embs_norm, recon_s, qloss_s, dup_cnt = pl.pallas_call(
        _fused_body,
        grid=grid,
        in_specs=in_specs,
        out_specs=out_specs,
        out_shape=out_shape,
        scratch_shapes=scratch_shapes,
    )(*args)

    recon_mean = recon_s[0, 0] / B
    qloss_mean = qloss_s[0, 0] / B
    loss = recon_mean + qloss_mean
    p_unique = (B - dup_cnt[0, 0]) / B
    return (loss, recon_mean, qloss_mean, embs_norm, p_unique)
